# baseline (speedup 1.0000x reference)
import functools
import math

import jax
import jax.numpy as jnp
from jax import lax
from jax.experimental import pallas as pl
from jax.experimental.pallas import tpu as pltpu
from jax.experimental.pallas import tpu_sc as plsc

NUM_SLOTS = 100000
D = 64
B = 1024

TM = 2048
NJ = (NUM_SLOTS + TM - 1) // TM
MP = NJ * TM
BB = 64
NB = B // BB

NW = 32
BPW = B // NW
SLICE = NUM_SLOTS // NW
CH = 625
NCH = SLICE // CH

_LN9 = math.log(0.9)


DA = 72
NBUF = 4
LASTW = NUM_SLOTS - (NJ - 1) * TM


def _attn_body(q2_ref, mem_hbm, attn_hbm, tail_hbm, rv_ref,
               memt_s, sbuf, inv_s, rv_s, obuf, mbuf, msem, osem):
    b = pl.program_id(0)
    j = pl.program_id(1)

    @pl.when(b == 0)
    def _stage_tile():
        @pl.when(j < NJ - 1)
        def _stage_full():
            pltpu.make_async_copy(
                mem_hbm.at[pl.ds(j * TM, TM), :], mbuf, msem).start()
            pltpu.make_async_copy(
                mem_hbm.at[pl.ds(j * TM, TM), :], mbuf, msem).wait()

        @pl.when(j == NJ - 1)
        def _stage_part():
            pltpu.make_async_copy(
                mem_hbm.at[pl.ds(j * TM, LASTW), :],
                mbuf.at[pl.ds(0, LASTW), :], msem).start()
            pltpu.make_async_copy(
                mem_hbm.at[pl.ds(j * TM, LASTW), :],
                mbuf.at[pl.ds(0, LASTW), :], msem).wait()
            mbuf[pl.ds(LASTW, TM - LASTW), :] = jnp.zeros(
                (TM - LASTW, D), jnp.float32)

        ii = lax.broadcasted_iota(jnp.int32, (D, D), 0)
        jj = lax.broadcasted_iota(jnp.int32, (D, D), 1)
        eye = (ii == jj).astype(jnp.float32)
        memt_s[:D, pl.ds(j * TM, TM)] = lax.dot_general(
            eye, mbuf[...], (((1,), (1,)), ((), ())),
            precision=lax.Precision.HIGHEST,
            preferred_element_type=jnp.float32)
        col = j * TM + lax.broadcasted_iota(jnp.int32, (1, TM), 1)
        memt_s[D:D + 1, pl.ds(j * TM, TM)] = jnp.where(
            col < NUM_SLOTS, 1.0, 0.0)
        memt_s[D + 1:DA, pl.ds(j * TM, TM)] = jnp.zeros(
            (DA - D - 1, TM), jnp.float32)

    @pl.when(j == 0)
    def _block_boundary():
        inv_s[...] = 1.0 / rv_s[:, D:D + 1]

        @pl.when(b > 0)
        def _emit_rv():
            rv_ref[...] = rv_s[:, :D] * inv_s[...]

        rv_s[...] = jnp.zeros_like(rv_s)

    wr = b * NJ + j
    t = wr % NBUF

    def _attn_desc(slot, jj, bb):
        return pltpu.make_async_copy(
            obuf.at[slot],
            attn_hbm.at[pl.ds((bb - 1) * BB, BB), pl.ds(jj * TM, TM)],
            osem.at[slot])

    def _tail_desc(slot, bb):
        return pltpu.make_async_copy(
            obuf.at[slot],
            tail_hbm.at[pl.ds((bb - 1) * BB, BB), :],
            osem.at[slot])

    @pl.when(b > 0)
    def _write_prev():
        @pl.when(wr >= NJ + NBUF)
        def _reclaim():
            _attn_desc(t, 0, 1).wait()

        obuf[t] = sbuf[:, pl.ds(j * TM, TM)] * inv_s[...]

        @pl.when(j < NJ - 1)
        def _start_main():
            _attn_desc(t, j, b).start()

        @pl.when(j == NJ - 1)
        def _start_tail():
            _tail_desc(t, b).start()

    @pl.when(b < NB)
    def _compute():
        mem_all = memt_s[:, pl.ds(j * TM, TM)]
        s = jnp.dot(q2_ref[...], mem_all[:D],
                    preferred_element_type=jnp.float32)
        e = jnp.exp(s)
        sbuf[:, pl.ds(j * TM, TM)] = e
        rv_s[...] += lax.dot_general(e, mem_all, (((1,), (1,)), ((), ())),
                                     preferred_element_type=jnp.float32)

    @pl.when((b == NB) & (j == NJ - 1))
    def _drain():
        for k in range(NBUF):
            _attn_desc(k, 0, 1).wait()


def _attention(q2, memory):
    return pl.pallas_call(
        _attn_body,
        grid=(NB + 1, NJ),
        in_specs=[
            pl.BlockSpec((BB, D), lambda b, j: (jnp.minimum(b, NB - 1), 0)),
            pl.BlockSpec(memory_space=pl.ANY),
        ],
        out_specs=[
            pl.BlockSpec(memory_space=pl.ANY),
            pl.BlockSpec(memory_space=pl.ANY),
            pl.BlockSpec((BB, D), lambda b, j: (jnp.maximum(b - 1, 0), 0)),
        ],
        out_shape=[
            jax.ShapeDtypeStruct((B, NUM_SLOTS), jnp.float32),
            jax.ShapeDtypeStruct((B, TM), jnp.float32),
            jax.ShapeDtypeStruct((B, D), jnp.float32),
        ],
        scratch_shapes=[
            pltpu.VMEM((DA, MP), jnp.float32),
            pltpu.VMEM((BB, MP), jnp.float32),
            pltpu.VMEM((BB, 1), jnp.float32),
            pltpu.VMEM((BB, DA), jnp.float32),
            pltpu.VMEM((NBUF, BB, TM), jnp.float32),
            pltpu.VMEM((TM, D), jnp.float32),
            pltpu.SemaphoreType.DMA,
            pltpu.SemaphoreType.DMA((NBUF,)),
        ],
    )(q2, memory)


def _tailmerge_body(tail_ref, attn_in_ref, attn_out_ref):
    del attn_in_ref
    attn_out_ref[...] = tail_ref[...]


def _tailmerge(attn_main, tail):
    return pl.pallas_call(
        _tailmerge_body,
        grid=(NB,),
        in_specs=[
            pl.BlockSpec((BB, TM), lambda b: (b, 0)),
            pl.BlockSpec(memory_space=pl.ANY),
        ],
        out_specs=pl.BlockSpec((BB, TM), lambda b: (b, NJ - 1)),
        out_shape=jax.ShapeDtypeStruct((B, NUM_SLOTS), jnp.float32),
        input_output_aliases={1: 0},
    )(tail, attn_main)


def _combine_body(value_ref, wg_ref, bg_ref, ac_ref, ar_ref, orig_ref,
                  query_ref, wq_ref, bq_ref, wk_ref,
                  rows_ref, q2_ref):
    q = jnp.dot(query_ref[...], wq_ref[...].T,
                preferred_element_type=jnp.float32) + bq_ref[...]
    q2_ref[...] = jnp.dot(q, wk_ref[...],
                          preferred_element_type=jnp.float32) * 0.125
    v = value_ref[...]
    g = jax.nn.sigmoid(jnp.sum(v * wg_ref[...], axis=1, keepdims=True)
                       + bg_ref[...])
    ac = ac_ref[...]
    ar = ar_ref[...]
    eq = ac == ar
    ef = eq.astype(jnp.float32)
    ii = lax.broadcasted_iota(jnp.int32, (B, B), 0)
    jj = lax.broadcasted_iota(jnp.int32, (B, B), 1)
    r = jnp.sum(jnp.where(eq & (jj > ii), 1.0, 0.0), axis=1, keepdims=True)
    c = jnp.sum(ef, axis=1, keepdims=True)
    coef = 0.1 * jnp.exp(r * _LN9) * g
    contrib = coef * v
    combined = lax.dot_general(ef, contrib, (((1,), (0,)), ((), ())),
                               precision=lax.Precision.HIGHEST,
                               preferred_element_type=jnp.float32)
    rows = jnp.exp(c * _LN9) * orig_ref[...] + combined
    rows_ref[...] = rows


def _combine(value, Wg2, bg2, addr_c, addr_r, orig, query, Wq, bq2, Wk):
    return pl.pallas_call(
        _combine_body,
        out_shape=[
            jax.ShapeDtypeStruct((B, D), jnp.float32),
            jax.ShapeDtypeStruct((B, D), jnp.float32),
        ],
    )(value, Wg2, bg2, addr_c, addr_r, orig, query, Wq, bq2, Wk)


def _sc_gather(memory, addr):
    mesh = plsc.VectorSubcoreMesh(core_axis_name="c", subcore_axis_name="s")

    @functools.partial(
        pl.kernel, mesh=mesh,
        out_type=jax.ShapeDtypeStruct((B, D), jnp.float32),
        compiler_params=pltpu.CompilerParams(use_tc_tiling_on_sc=False),
        scratch_types=[
            pltpu.VMEM((BPW,), jnp.int32),
            pltpu.VMEM((BPW, D), jnp.float32),
            pltpu.SemaphoreType.DMA,
        ],
    )
    def k(mem_hbm, idx_hbm, out_hbm, idx_v, rows_v, sem):
        wid = lax.axis_index("s") * 2 + lax.axis_index("c")
        base = wid * BPW
        pltpu.sync_copy(idx_hbm.at[pl.ds(base, BPW)], idx_v)
        pltpu.async_copy(mem_hbm.at[idx_v], rows_v, sem).wait()
        pltpu.sync_copy(rows_v, out_hbm.at[pl.ds(base, BPW)])

    return k(memory, addr)


def _sc_write(memory, addr, rows):
    mesh = plsc.VectorSubcoreMesh(core_axis_name="c", subcore_axis_name="s")

    @functools.partial(
        pl.kernel, mesh=mesh,
        out_type=jax.ShapeDtypeStruct((NUM_SLOTS + NW, D), jnp.float32),
        compiler_params=pltpu.CompilerParams(use_tc_tiling_on_sc=False),
        scratch_types=[
            pltpu.VMEM((CH, D), jnp.float32),
            pltpu.VMEM((B,), jnp.int32),
            pltpu.VMEM((8, 128), jnp.int32),
            pltpu.VMEM((B, D), jnp.float32),
            pltpu.SemaphoreType.DMA,
        ],
    )
    def k(mem_hbm, addr_hbm, rows_hbm, out_hbm,
          cbuf, addr_v, idx_v, rows_v, sem):
        wid = lax.axis_index("s") * 2 + lax.axis_index("c")
        lo = wid * SLICE
        for ci in range(NCH):
            pltpu.sync_copy(mem_hbm.at[pl.ds(lo + ci * CH, CH)], cbuf)
            pltpu.sync_copy(cbuf, out_hbm.at[pl.ds(lo + ci * CH, CH)])
        pltpu.sync_copy(rows_hbm, rows_v)
        pltpu.sync_copy(addr_hbm, addr_v)
        for i in range(B // 16):
            a = addr_v[pl.ds(i * 16, 16)]
            inr = (a >= lo) & (a < lo + SLICE)
            idx_v[i // 8, pl.ds((i % 8) * 16, 16)] = jnp.where(
                inr, a, NUM_SLOTS + wid)
        cps = [
            pltpu.async_copy(rows_v.at[pl.ds(ci * 128, 128)],
                             out_hbm.at[idx_v.at[ci]], sem)
            for ci in range(8)
        ]
        for cp in cps:
            cp.wait()

    return k(memory, addr, rows)


def kernel(query, value, location_id, memory, Wq, bq, Wk, bk, Wg, bg):
    del bk
    addr = (location_id.astype(jnp.int32)) % NUM_SLOTS
    bq2 = bq.reshape(1, D)
    Wg2 = Wg.reshape(1, D)
    bg2 = bg.reshape(1, 1)
    addr_c = addr.reshape(B, 1)
    addr_r = addr.reshape(1, B)

    orig = _sc_gather(memory, addr)
    rows, q2 = _combine(value, Wg2, bg2, addr_c, addr_r, orig,
                        query, Wq, bq2, Wk)
    attn_main, tail, rv = _attention(q2, memory)
    attn = _tailmerge(attn_main, tail)
    new_memory = _sc_write(memory, addr, rows)[:NUM_SLOTS]
    return (rv, attn, new_memory)


# speedup vs baseline: 1.0365x; 1.0365x over previous
import functools
import math

import jax
import jax.numpy as jnp
from jax import lax
from jax.experimental import pallas as pl
from jax.experimental.pallas import tpu as pltpu
from jax.experimental.pallas import tpu_sc as plsc

NUM_SLOTS = 100000
D = 64
B = 1024

TM = 2048
NJ = (NUM_SLOTS + TM - 1) // TM
MP = NJ * TM
BB = 64
NB = B // BB

NW = 32
BPW = B // NW
SLICE = NUM_SLOTS // NW
CH = 625
NCH = SLICE // CH

_LN9 = math.log(0.9)


DA = 72
NBUF = 4
LASTW = NUM_SLOTS - (NJ - 1) * TM


def _attn_body(q2_ref, memt_hbm, attn_hbm, tail_hbm, rv_ref,
               memt_s, sbuf, inv_s, rv_s, obuf, msem, osem):
    b = pl.program_id(0)
    j = pl.program_id(1)

    @pl.when((b == 0) & (j == 0))
    def _stage_memory():
        cp = pltpu.make_async_copy(memt_hbm, memt_s, msem)
        cp.start()
        cp.wait()

    @pl.when(j == 0)
    def _block_boundary():
        inv_s[...] = 1.0 / rv_s[:, D:D + 1]

        @pl.when(b > 0)
        def _emit_rv():
            rv_ref[...] = rv_s[:, :D] * inv_s[...]

        rv_s[...] = jnp.zeros_like(rv_s)

    wr = b * NJ + j
    t = wr % NBUF

    def _attn_desc(slot, jj, bb):
        return pltpu.make_async_copy(
            obuf.at[slot],
            attn_hbm.at[pl.ds((bb - 1) * BB, BB), pl.ds(jj * TM, TM)],
            osem.at[slot])

    def _tail_desc(slot, bb):
        return pltpu.make_async_copy(
            obuf.at[slot],
            tail_hbm.at[pl.ds((bb - 1) * BB, BB), :],
            osem.at[slot])

    @pl.when(b > 0)
    def _write_prev():
        @pl.when(wr >= NJ + NBUF)
        def _reclaim():
            _attn_desc(t, 0, 1).wait()

        obuf[t] = sbuf[:, pl.ds(j * TM, TM)] * inv_s[...]

        @pl.when(j < NJ - 1)
        def _start_main():
            _attn_desc(t, j, b).start()

        @pl.when(j == NJ - 1)
        def _start_tail():
            _tail_desc(t, b).start()

    @pl.when(b < NB)
    def _compute():
        mem_all = memt_s[:, pl.ds(j * TM, TM)]
        s = jnp.dot(q2_ref[...], mem_all[:D],
                    preferred_element_type=jnp.float32)
        e = jnp.exp(s)
        sbuf[:, pl.ds(j * TM, TM)] = e
        rv_s[...] += lax.dot_general(e, mem_all, (((1,), (1,)), ((), ())),
                                     preferred_element_type=jnp.float32)

    @pl.when((b == NB) & (j == NJ - 1))
    def _drain():
        for k in range(NBUF):
            _attn_desc(k, 0, 1).wait()


def _transpose_body(mem_ref, out_ref):
    j = pl.program_id(0)
    ii = lax.broadcasted_iota(jnp.int32, (D, D), 0)
    jj = lax.broadcasted_iota(jnp.int32, (D, D), 1)
    eye = (ii == jj).astype(jnp.float32)
    t = lax.dot_general(eye, mem_ref[...], (((1,), (1,)), ((), ())),
                        precision=lax.Precision.HIGHEST,
                        preferred_element_type=jnp.float32)
    col = j * TM + lax.broadcasted_iota(jnp.int32, (1, TM), 1)
    valid = col < NUM_SLOTS
    out_ref[:D] = jnp.where(valid, t, 0.0)
    out_ref[D:D + 1] = jnp.where(valid, 1.0, 0.0)
    out_ref[D + 1:DA] = jnp.zeros((DA - D - 1, TM), jnp.float32)


def _transpose(memory):
    return pl.pallas_call(
        _transpose_body,
        grid=(NJ,),
        in_specs=[pl.BlockSpec((TM, D), lambda j: (j, 0))],
        out_specs=pl.BlockSpec((DA, TM), lambda j: (0, j)),
        out_shape=jax.ShapeDtypeStruct((DA, MP), jnp.float32),
    )(memory)


def _attention(q2, memT_aug):
    return pl.pallas_call(
        _attn_body,
        grid=(NB + 1, NJ),
        in_specs=[
            pl.BlockSpec((BB, D), lambda b, j: (jnp.minimum(b, NB - 1), 0)),
            pl.BlockSpec(memory_space=pl.ANY),
        ],
        out_specs=[
            pl.BlockSpec(memory_space=pl.ANY),
            pl.BlockSpec(memory_space=pl.ANY),
            pl.BlockSpec((BB, D), lambda b, j: (jnp.maximum(b - 1, 0), 0)),
        ],
        out_shape=[
            jax.ShapeDtypeStruct((B, NUM_SLOTS), jnp.float32),
            jax.ShapeDtypeStruct((B, TM), jnp.float32),
            jax.ShapeDtypeStruct((B, D), jnp.float32),
        ],
        scratch_shapes=[
            pltpu.VMEM((DA, MP), jnp.float32),
            pltpu.VMEM((BB, MP), jnp.float32),
            pltpu.VMEM((BB, 1), jnp.float32),
            pltpu.VMEM((BB, DA), jnp.float32),
            pltpu.VMEM((NBUF, BB, TM), jnp.float32),
            pltpu.SemaphoreType.DMA,
            pltpu.SemaphoreType.DMA((NBUF,)),
        ],
    )(q2, memT_aug)


def _tailmerge_body(tail_ref, attn_in_ref, attn_out_ref):
    del attn_in_ref
    attn_out_ref[...] = tail_ref[...]


def _tailmerge(attn_main, tail):
    return pl.pallas_call(
        _tailmerge_body,
        grid=(NB,),
        in_specs=[
            pl.BlockSpec((BB, TM), lambda b: (b, 0)),
            pl.BlockSpec(memory_space=pl.ANY),
        ],
        out_specs=pl.BlockSpec((BB, TM), lambda b: (b, NJ - 1)),
        out_shape=jax.ShapeDtypeStruct((B, NUM_SLOTS), jnp.float32),
        input_output_aliases={1: 0},
    )(tail, attn_main)


def _combine_body(value_ref, wg_ref, bg_ref, ac_ref, ar_ref, orig_ref,
                  query_ref, wq_ref, bq_ref, wk_ref,
                  rows_ref, q2_ref):
    q = jnp.dot(query_ref[...], wq_ref[...].T,
                preferred_element_type=jnp.float32) + bq_ref[...]
    q2_ref[...] = jnp.dot(q, wk_ref[...],
                          preferred_element_type=jnp.float32) * 0.125
    v = value_ref[...]
    g = jax.nn.sigmoid(jnp.sum(v * wg_ref[...], axis=1, keepdims=True)
                       + bg_ref[...])
    ac = ac_ref[...]
    ar = ar_ref[...]
    eq = ac == ar
    ef = eq.astype(jnp.float32)
    ii = lax.broadcasted_iota(jnp.int32, (B, B), 0)
    jj = lax.broadcasted_iota(jnp.int32, (B, B), 1)
    r = jnp.sum(jnp.where(eq & (jj > ii), 1.0, 0.0), axis=1, keepdims=True)
    c = jnp.sum(ef, axis=1, keepdims=True)
    coef = 0.1 * jnp.exp(r * _LN9) * g
    contrib = coef * v
    combined = lax.dot_general(ef, contrib, (((1,), (0,)), ((), ())),
                               precision=lax.Precision.HIGHEST,
                               preferred_element_type=jnp.float32)
    rows = jnp.exp(c * _LN9) * orig_ref[...] + combined
    rows_ref[...] = rows


def _combine(value, Wg2, bg2, addr_c, addr_r, orig, query, Wq, bq2, Wk):
    return pl.pallas_call(
        _combine_body,
        out_shape=[
            jax.ShapeDtypeStruct((B, D), jnp.float32),
            jax.ShapeDtypeStruct((B, D), jnp.float32),
        ],
    )(value, Wg2, bg2, addr_c, addr_r, orig, query, Wq, bq2, Wk)


def _sc_gather(memory, addr):
    mesh = plsc.VectorSubcoreMesh(core_axis_name="c", subcore_axis_name="s")

    @functools.partial(
        pl.kernel, mesh=mesh,
        out_type=jax.ShapeDtypeStruct((B, D), jnp.float32),
        compiler_params=pltpu.CompilerParams(use_tc_tiling_on_sc=False),
        scratch_types=[
            pltpu.VMEM((BPW,), jnp.int32),
            pltpu.VMEM((BPW, D), jnp.float32),
            pltpu.SemaphoreType.DMA,
        ],
    )
    def k(mem_hbm, idx_hbm, out_hbm, idx_v, rows_v, sem):
        wid = lax.axis_index("s") * 2 + lax.axis_index("c")
        base = wid * BPW
        pltpu.sync_copy(idx_hbm.at[pl.ds(base, BPW)], idx_v)
        pltpu.async_copy(mem_hbm.at[idx_v], rows_v, sem).wait()
        pltpu.sync_copy(rows_v, out_hbm.at[pl.ds(base, BPW)])

    return k(memory, addr)


def _sc_write(memory, addr, rows):
    mesh = plsc.VectorSubcoreMesh(core_axis_name="c", subcore_axis_name="s")

    @functools.partial(
        pl.kernel, mesh=mesh,
        out_type=jax.ShapeDtypeStruct((NUM_SLOTS + NW, D), jnp.float32),
        compiler_params=pltpu.CompilerParams(use_tc_tiling_on_sc=False),
        scratch_types=[
            pltpu.VMEM((CH, D), jnp.float32),
            pltpu.VMEM((B,), jnp.int32),
            pltpu.VMEM((8, 128), jnp.int32),
            pltpu.VMEM((B, D), jnp.float32),
            pltpu.SemaphoreType.DMA,
        ],
    )
    def k(mem_hbm, addr_hbm, rows_hbm, out_hbm,
          cbuf, addr_v, idx_v, rows_v, sem):
        wid = lax.axis_index("s") * 2 + lax.axis_index("c")
        lo = wid * SLICE
        for ci in range(NCH):
            pltpu.sync_copy(mem_hbm.at[pl.ds(lo + ci * CH, CH)], cbuf)
            pltpu.sync_copy(cbuf, out_hbm.at[pl.ds(lo + ci * CH, CH)])
        pltpu.sync_copy(rows_hbm, rows_v)
        pltpu.sync_copy(addr_hbm, addr_v)
        for i in range(B // 16):
            a = addr_v[pl.ds(i * 16, 16)]
            inr = (a >= lo) & (a < lo + SLICE)
            idx_v[i // 8, pl.ds((i % 8) * 16, 16)] = jnp.where(
                inr, a, NUM_SLOTS + wid)
        cps = [
            pltpu.async_copy(rows_v.at[pl.ds(ci * 128, 128)],
                             out_hbm.at[idx_v.at[ci]], sem)
            for ci in range(8)
        ]
        for cp in cps:
            cp.wait()

    return k(memory, addr, rows)


def kernel(query, value, location_id, memory, Wq, bq, Wk, bk, Wg, bg):
    del bk
    addr = (location_id.astype(jnp.int32)) % NUM_SLOTS
    bq2 = bq.reshape(1, D)
    Wg2 = Wg.reshape(1, D)
    bg2 = bg.reshape(1, 1)
    addr_c = addr.reshape(B, 1)
    addr_r = addr.reshape(1, B)

    orig = _sc_gather(memory, addr)
    rows, q2 = _combine(value, Wg2, bg2, addr_c, addr_r, orig,
                        query, Wq, bq2, Wk)
    memT_aug = _transpose(memory)
    attn_main, tail, rv = _attention(q2, memT_aug)
    attn = _tailmerge(attn_main, tail)
    new_memory = _sc_write(memory, addr, rows)[:NUM_SLOTS]
    return (rv, attn, new_memory)


# speedup vs baseline: 1.0924x; 1.0539x over previous
import functools
import math

import jax
import jax.numpy as jnp
from jax import lax
from jax.experimental import pallas as pl
from jax.experimental.pallas import tpu as pltpu
from jax.experimental.pallas import tpu_sc as plsc

NUM_SLOTS = 100000
D = 64
B = 1024

TM = 2048
NJ = (NUM_SLOTS + TM - 1) // TM
MP = NJ * TM
BB = 64
NB = B // BB

NW = 32
BPW = B // NW
SLICE = NUM_SLOTS // NW
CH = 625
NCH = SLICE // CH

_LN9 = math.log(0.9)


DA = 72
NBUF = 4
LASTW = NUM_SLOTS - (NJ - 1) * TM


def _attn_body(q2_ref, memt_hbm, attn_hbm, tail_hbm, rv_ref,
               memt_s, sbuf, inv_s, rv_s, obuf, msem, osem):
    b = pl.program_id(0)
    j = pl.program_id(1)

    @pl.when((b == 0) & (j == 0))
    def _stage_memory():
        cp = pltpu.make_async_copy(memt_hbm, memt_s, msem)
        cp.start()
        cp.wait()

    @pl.when(j == 0)
    def _block_boundary():
        inv_s[...] = 1.0 / rv_s[:, D:D + 1]

        @pl.when(b > 0)
        def _emit_rv():
            rv_ref[...] = rv_s[:, :D] * inv_s[...]

        rv_s[...] = jnp.zeros_like(rv_s)

    wr = b * NJ + j
    t = wr % NBUF

    def _attn_desc(slot, jj, bb):
        return pltpu.make_async_copy(
            obuf.at[slot],
            attn_hbm.at[pl.ds((bb - 1) * BB, BB), pl.ds(jj * TM, TM)],
            osem.at[slot])

    def _tail_desc(slot, bb):
        return pltpu.make_async_copy(
            obuf.at[slot],
            tail_hbm.at[pl.ds((bb - 1) * BB, BB), :],
            osem.at[slot])

    @pl.when(b > 0)
    def _write_prev():
        @pl.when(wr >= NJ + NBUF)
        def _reclaim():
            _attn_desc(t, 0, 1).wait()

        obuf[t] = sbuf[:, pl.ds(j * TM, TM)] * inv_s[...]

        @pl.when(j < NJ - 1)
        def _start_main():
            _attn_desc(t, j, b).start()

        @pl.when(j == NJ - 1)
        def _start_tail():
            _tail_desc(t, b).start()

    @pl.when(b < NB)
    def _compute():
        mem_all = memt_s[:, pl.ds(j * TM, TM)]
        s = jnp.dot(q2_ref[...], mem_all[:D],
                    preferred_element_type=jnp.float32)
        e = jnp.exp(s)
        sbuf[:, pl.ds(j * TM, TM)] = e
        rv_s[...] += lax.dot_general(e, mem_all, (((1,), (1,)), ((), ())),
                                     preferred_element_type=jnp.float32)

    @pl.when((b == NB) & (j == NJ - 1))
    def _drain():
        for k in range(NBUF):
            _attn_desc(k, 0, 1).wait()


def _transpose_body(mem_ref, out_ref):
    j = pl.program_id(0)
    ii = lax.broadcasted_iota(jnp.int32, (D, D), 0)
    jj = lax.broadcasted_iota(jnp.int32, (D, D), 1)
    eye = (ii == jj).astype(jnp.float32)
    t = lax.dot_general(eye, mem_ref[...], (((1,), (1,)), ((), ())),
                        precision=lax.Precision.HIGHEST,
                        preferred_element_type=jnp.float32)
    col = j * TM + lax.broadcasted_iota(jnp.int32, (1, TM), 1)
    valid = col < NUM_SLOTS
    out_ref[:D] = jnp.where(valid, t, 0.0)
    out_ref[D:D + 1] = jnp.where(valid, 1.0, 0.0)
    out_ref[D + 1:DA] = jnp.zeros((DA - D - 1, TM), jnp.float32)


def _transpose(memory):
    return pl.pallas_call(
        _transpose_body,
        grid=(NJ,),
        in_specs=[pl.BlockSpec((TM, D), lambda j: (j, 0))],
        out_specs=pl.BlockSpec((DA, TM), lambda j: (0, j)),
        out_shape=jax.ShapeDtypeStruct((DA, MP), jnp.float32),
    )(memory)


def _attention(q2, memT_aug):
    return pl.pallas_call(
        _attn_body,
        grid=(NB + 1, NJ),
        in_specs=[
            pl.BlockSpec((BB, D), lambda b, j: (jnp.minimum(b, NB - 1), 0)),
            pl.BlockSpec(memory_space=pl.ANY),
        ],
        out_specs=[
            pl.BlockSpec(memory_space=pl.ANY),
            pl.BlockSpec(memory_space=pl.ANY),
            pl.BlockSpec((BB, D), lambda b, j: (jnp.maximum(b - 1, 0), 0)),
        ],
        out_shape=[
            jax.ShapeDtypeStruct((B, NUM_SLOTS), jnp.float32),
            jax.ShapeDtypeStruct((B, TM), jnp.float32),
            jax.ShapeDtypeStruct((B, D), jnp.float32),
        ],
        scratch_shapes=[
            pltpu.VMEM((DA, MP), jnp.float32),
            pltpu.VMEM((BB, MP), jnp.float32),
            pltpu.VMEM((BB, 1), jnp.float32),
            pltpu.VMEM((BB, DA), jnp.float32),
            pltpu.VMEM((NBUF, BB, TM), jnp.float32),
            pltpu.SemaphoreType.DMA,
            pltpu.SemaphoreType.DMA((NBUF,)),
        ],
    )(q2, memT_aug)


def _tailmerge_body(tail_ref, attn_in_ref, attn_out_ref):
    del attn_in_ref
    attn_out_ref[...] = tail_ref[...]


def _tailmerge(attn_main, tail):
    return pl.pallas_call(
        _tailmerge_body,
        grid=(NB,),
        in_specs=[
            pl.BlockSpec((BB, TM), lambda b: (b, 0)),
            pl.BlockSpec(memory_space=pl.ANY),
        ],
        out_specs=pl.BlockSpec((BB, TM), lambda b: (b, NJ - 1)),
        out_shape=jax.ShapeDtypeStruct((B, NUM_SLOTS), jnp.float32),
        input_output_aliases={1: 0},
    )(tail, attn_main)


def _combine_body(value_ref, wg_ref, bg_ref, ac_ref, ar_ref, orig_ref,
                  query_ref, wq_ref, bq_ref, wk_ref,
                  rows_ref, q2_ref):
    q = jnp.dot(query_ref[...], wq_ref[...].T,
                preferred_element_type=jnp.float32) + bq_ref[...]
    q2_ref[...] = jnp.dot(q, wk_ref[...],
                          preferred_element_type=jnp.float32) * 0.125
    v = value_ref[...]
    g = jax.nn.sigmoid(jnp.sum(v * wg_ref[...], axis=1, keepdims=True)
                       + bg_ref[...])
    ac = ac_ref[...]
    ar = ar_ref[...]
    eq = ac == ar
    ef = eq.astype(jnp.float32)
    ii = lax.broadcasted_iota(jnp.int32, (B, B), 0)
    jj = lax.broadcasted_iota(jnp.int32, (B, B), 1)
    r = jnp.sum(jnp.where(eq & (jj > ii), 1.0, 0.0), axis=1, keepdims=True)
    c = jnp.sum(ef, axis=1, keepdims=True)
    coef = 0.1 * jnp.exp(r * _LN9) * g
    contrib = coef * v
    combined = lax.dot_general(ef, contrib, (((1,), (0,)), ((), ())),
                               precision=lax.Precision.HIGHEST,
                               preferred_element_type=jnp.float32)
    rows = jnp.exp(c * _LN9) * orig_ref[...] + combined
    rows_ref[...] = rows


def _combine(value, Wg2, bg2, addr_c, addr_r, orig, query, Wq, bq2, Wk):
    return pl.pallas_call(
        _combine_body,
        out_shape=[
            jax.ShapeDtypeStruct((B, D), jnp.float32),
            jax.ShapeDtypeStruct((B, D), jnp.float32),
        ],
    )(value, Wg2, bg2, addr_c, addr_r, orig, query, Wq, bq2, Wk)


def _sc_gather(memory, addr):
    mesh = plsc.VectorSubcoreMesh(core_axis_name="c", subcore_axis_name="s")

    @functools.partial(
        pl.kernel, mesh=mesh,
        out_type=jax.ShapeDtypeStruct((B, D), jnp.float32),
        compiler_params=pltpu.CompilerParams(use_tc_tiling_on_sc=False),
        scratch_types=[
            pltpu.VMEM((BPW,), jnp.int32),
            pltpu.VMEM((BPW, D), jnp.float32),
            pltpu.SemaphoreType.DMA,
        ],
    )
    def k(mem_hbm, idx_hbm, out_hbm, idx_v, rows_v, sem):
        wid = lax.axis_index("s") * 2 + lax.axis_index("c")
        base = wid * BPW
        pltpu.sync_copy(idx_hbm.at[pl.ds(base, BPW)], idx_v)
        pltpu.async_copy(mem_hbm.at[idx_v], rows_v, sem).wait()
        pltpu.sync_copy(rows_v, out_hbm.at[pl.ds(base, BPW)])

    return k(memory, addr)


def _sc_write(memory, addr, rows):
    mesh = plsc.VectorSubcoreMesh(core_axis_name="c", subcore_axis_name="s")

    @functools.partial(
        pl.kernel, mesh=mesh,
        out_type=jax.ShapeDtypeStruct((NUM_SLOTS + NW, D), jnp.float32),
        compiler_params=pltpu.CompilerParams(use_tc_tiling_on_sc=False),
        scratch_types=[
            pltpu.VMEM((CH, D), jnp.float32),
            pltpu.VMEM((B,), jnp.int32),
            pltpu.VMEM((8, 128), jnp.int32),
            pltpu.VMEM((B, D), jnp.float32),
            pltpu.SemaphoreType.DMA,
        ],
    )
    def k(mem_hbm, addr_hbm, rows_hbm, out_hbm,
          cbuf, addr_v, idx_v, rows_v, sem):
        wid = lax.axis_index("s") * 2 + lax.axis_index("c")
        lo = wid * SLICE
        for ci in range(NCH):
            pltpu.sync_copy(mem_hbm.at[pl.ds(lo + ci * CH, CH)], cbuf)
            pltpu.sync_copy(cbuf, out_hbm.at[pl.ds(lo + ci * CH, CH)])
        pltpu.sync_copy(rows_hbm, rows_v)
        pltpu.sync_copy(addr_hbm, addr_v)
        for i in range(B // 16):
            a = addr_v[pl.ds(i * 16, 16)]
            inr = (a >= lo) & (a < lo + SLICE)
            idx_v[i // 8, pl.ds((i % 8) * 16, 16)] = jnp.where(
                inr, a, NUM_SLOTS + wid)
        cps = [
            pltpu.async_copy(rows_v.at[pl.ds(ci * 128, 128)],
                             out_hbm.at[idx_v.at[ci]], sem)
            for ci in range(8)
        ]
        for cp in cps:
            cp.wait()

    return k(memory, addr, rows)


def kernel(query, value, location_id, memory, Wq, bq, Wk, bk, Wg, bg):
    del bk
    addr = (location_id.astype(jnp.int32)) % NUM_SLOTS
    bq2 = bq.reshape(1, D)
    Wg2 = Wg.reshape(1, D)
    bg2 = bg.reshape(1, 1)
    addr_c = addr.reshape(B, 1)
    addr_r = addr.reshape(1, B)

    orig = _sc_gather(memory, addr)
    rows, q2 = _combine(value, Wg2, bg2, addr_c, addr_r, orig,
                        query, Wq, bq2, Wk)
    memT_aug = jnp.pad(
        jnp.concatenate(
            [memory.T,
             jnp.ones((1, NUM_SLOTS), jnp.float32),
             jnp.zeros((DA - D - 1, NUM_SLOTS), jnp.float32)], axis=0),
        ((0, 0), (0, MP - NUM_SLOTS)))
    attn_main, tail, rv = _attention(q2, memT_aug)
    attn = _tailmerge(attn_main, tail)
    new_memory = _sc_write(memory, addr, rows)[:NUM_SLOTS]
    return (rv, attn, new_memory)


# speedup vs baseline: 1.0936x; 1.0011x over previous
import functools
import math

import jax
import jax.numpy as jnp
from jax import lax
from jax.experimental import pallas as pl
from jax.experimental.pallas import tpu as pltpu
from jax.experimental.pallas import tpu_sc as plsc

NUM_SLOTS = 100000
D = 64
B = 1024

TM = 2048
NJ = (NUM_SLOTS + TM - 1) // TM
MP = NJ * TM
BB = 64
NB = B // BB

NW = 32
BPW = B // NW
SLICE = NUM_SLOTS // NW
CH = 625
NCH = SLICE // CH

_LN9 = math.log(0.9)


DA = 72
NBUF = 4
LASTW = NUM_SLOTS - (NJ - 1) * TM


def _attn_body(q2_ref, memt_hbm, attn_hbm, tail_hbm, rv_ref,
               memt_s, sbuf, inv_s, rv_s, obuf, msem, osem):
    b = pl.program_id(0)
    j = pl.program_id(1)

    @pl.when((b == 0) & (j == 0))
    def _stage_memory():
        cp = pltpu.make_async_copy(memt_hbm, memt_s, msem)
        cp.start()
        cp.wait()

    @pl.when(j == 0)
    def _block_boundary():
        inv_s[...] = 1.0 / rv_s[:, D:D + 1]

        @pl.when(b > 0)
        def _emit_rv():
            rv_ref[...] = rv_s[:, :D] * inv_s[...]

        rv_s[...] = jnp.zeros_like(rv_s)

    wr = b * NJ + j
    t = wr % NBUF

    def _attn_desc(slot, jj, bb):
        return pltpu.make_async_copy(
            obuf.at[slot],
            attn_hbm.at[pl.ds((bb - 1) * BB, BB), pl.ds(jj * TM, TM)],
            osem.at[slot])

    def _tail_desc(slot, bb):
        return pltpu.make_async_copy(
            obuf.at[slot],
            tail_hbm.at[pl.ds((bb - 1) * BB, BB), :],
            osem.at[slot])

    @pl.when(b > 0)
    def _write_prev():
        @pl.when(wr >= NJ + NBUF)
        def _reclaim():
            _attn_desc(t, 0, 1).wait()

        obuf[t] = sbuf[:, pl.ds(j * TM, TM)] * inv_s[...]

        @pl.when(j < NJ - 1)
        def _start_main():
            _attn_desc(t, j, b).start()

        @pl.when(j == NJ - 1)
        def _start_tail():
            _tail_desc(t, b).start()

    @pl.when(b < NB)
    def _compute():
        mem_all = memt_s[:, pl.ds(j * TM, TM)]
        s = jnp.dot(q2_ref[...], mem_all[:D],
                    preferred_element_type=jnp.float32)
        e = jnp.exp(s)
        sbuf[:, pl.ds(j * TM, TM)] = e
        rv_s[...] += lax.dot_general(e, mem_all, (((1,), (1,)), ((), ())),
                                     preferred_element_type=jnp.float32)

    @pl.when((b == NB) & (j == NJ - 1))
    def _drain():
        for k in range(NBUF):
            _attn_desc(k, 0, 1).wait()


def _attention(q2, memT_aug):
    return pl.pallas_call(
        _attn_body,
        grid=(NB + 1, NJ),
        in_specs=[
            pl.BlockSpec((BB, D), lambda b, j: (jnp.minimum(b, NB - 1), 0)),
            pl.BlockSpec(memory_space=pl.ANY),
        ],
        out_specs=[
            pl.BlockSpec(memory_space=pl.ANY),
            pl.BlockSpec(memory_space=pl.ANY),
            pl.BlockSpec((BB, D), lambda b, j: (jnp.maximum(b - 1, 0), 0)),
        ],
        out_shape=[
            jax.ShapeDtypeStruct((B, NUM_SLOTS), jnp.float32),
            jax.ShapeDtypeStruct((B, TM), jnp.float32),
            jax.ShapeDtypeStruct((B, D), jnp.float32),
        ],
        scratch_shapes=[
            pltpu.VMEM((DA, MP), jnp.float32),
            pltpu.VMEM((BB, MP), jnp.float32),
            pltpu.VMEM((BB, 1), jnp.float32),
            pltpu.VMEM((BB, DA), jnp.float32),
            pltpu.VMEM((NBUF, BB, TM), jnp.float32),
            pltpu.SemaphoreType.DMA,
            pltpu.SemaphoreType.DMA((NBUF,)),
        ],
    )(q2, memT_aug)


def _tailmerge_body(tail_ref, attn_in_ref, attn_out_ref):
    del attn_in_ref
    attn_out_ref[...] = tail_ref[...]


def _tailmerge(attn_main, tail):
    return pl.pallas_call(
        _tailmerge_body,
        grid=(NB,),
        in_specs=[
            pl.BlockSpec((BB, TM), lambda b: (b, 0)),
            pl.BlockSpec(memory_space=pl.ANY),
        ],
        out_specs=pl.BlockSpec((BB, TM), lambda b: (b, NJ - 1)),
        out_shape=jax.ShapeDtypeStruct((B, NUM_SLOTS), jnp.float32),
        input_output_aliases={1: 0},
    )(tail, attn_main)


def _combine_body(value_ref, wg_ref, bg_ref, ac_ref, ar_ref, orig_ref,
                  query_ref, wq_ref, bq_ref, wk_ref,
                  rows_ref, q2_ref):
    q = jnp.dot(query_ref[...], wq_ref[...].T,
                preferred_element_type=jnp.float32) + bq_ref[...]
    q2_ref[...] = jnp.dot(q, wk_ref[...],
                          preferred_element_type=jnp.float32) * 0.125
    v = value_ref[...]
    g = jax.nn.sigmoid(jnp.sum(v * wg_ref[...], axis=1, keepdims=True)
                       + bg_ref[...])
    ac = ac_ref[...]
    ar = ar_ref[...]
    eq = ac == ar
    ef = eq.astype(jnp.float32)
    ii = lax.broadcasted_iota(jnp.int32, (B, B), 0)
    jj = lax.broadcasted_iota(jnp.int32, (B, B), 1)
    r = jnp.sum(jnp.where(eq & (jj > ii), 1.0, 0.0), axis=1, keepdims=True)
    c = jnp.sum(ef, axis=1, keepdims=True)
    coef = 0.1 * jnp.exp(r * _LN9) * g
    contrib = coef * v
    combined = lax.dot_general(ef, contrib, (((1,), (0,)), ((), ())),
                               precision=lax.Precision.HIGHEST,
                               preferred_element_type=jnp.float32)
    rows = jnp.exp(c * _LN9) * orig_ref[...] + combined
    rows_ref[...] = rows


def _combine(value, Wg2, bg2, addr_c, addr_r, orig, query, Wq, bq2, Wk):
    return pl.pallas_call(
        _combine_body,
        out_shape=[
            jax.ShapeDtypeStruct((B, D), jnp.float32),
            jax.ShapeDtypeStruct((B, D), jnp.float32),
        ],
    )(value, Wg2, bg2, addr_c, addr_r, orig, query, Wq, bq2, Wk)


def _sc_gather(memory, addr):
    mesh = plsc.VectorSubcoreMesh(core_axis_name="c", subcore_axis_name="s")

    @functools.partial(
        pl.kernel, mesh=mesh,
        out_type=jax.ShapeDtypeStruct((B, D), jnp.float32),
        compiler_params=pltpu.CompilerParams(use_tc_tiling_on_sc=False),
        scratch_types=[
            pltpu.VMEM((BPW,), jnp.int32),
            pltpu.VMEM((BPW, D), jnp.float32),
            pltpu.SemaphoreType.DMA,
        ],
    )
    def k(mem_hbm, idx_hbm, out_hbm, idx_v, rows_v, sem):
        wid = lax.axis_index("s") * 2 + lax.axis_index("c")
        base = wid * BPW
        pltpu.sync_copy(idx_hbm.at[pl.ds(base, BPW)], idx_v)
        pltpu.async_copy(mem_hbm.at[idx_v], rows_v, sem).wait()
        pltpu.sync_copy(rows_v, out_hbm.at[pl.ds(base, BPW)])

    return k(memory, addr)


def _sc_write(memory, addr, rows):
    mesh = plsc.VectorSubcoreMesh(core_axis_name="c", subcore_axis_name="s")

    @functools.partial(
        pl.kernel, mesh=mesh,
        out_type=jax.ShapeDtypeStruct((NUM_SLOTS + NW, D), jnp.float32),
        compiler_params=pltpu.CompilerParams(use_tc_tiling_on_sc=False),
        scratch_types=[
            pltpu.VMEM((CH, D), jnp.float32),
            pltpu.VMEM((B,), jnp.int32),
            pltpu.VMEM((8, 128), jnp.int32),
            pltpu.VMEM((B, D), jnp.float32),
            pltpu.SemaphoreType.DMA,
        ],
    )
    def k(mem_hbm, addr_hbm, rows_hbm, out_hbm,
          cbuf, addr_v, idx_v, rows_v, sem):
        wid = lax.axis_index("s") * 2 + lax.axis_index("c")
        lo = wid * SLICE
        for ci in range(NCH):
            pltpu.sync_copy(mem_hbm.at[pl.ds(lo + ci * CH, CH)], cbuf)
            pltpu.sync_copy(cbuf, out_hbm.at[pl.ds(lo + ci * CH, CH)])
        pltpu.sync_copy(rows_hbm, rows_v)
        pltpu.sync_copy(addr_hbm, addr_v)
        for i in range(B // 16):
            a = addr_v[pl.ds(i * 16, 16)]
            inr = (a >= lo) & (a < lo + SLICE)
            idx_v[i // 8, pl.ds((i % 8) * 16, 16)] = jnp.where(
                inr, a, NUM_SLOTS + wid)
        cps = [
            pltpu.async_copy(rows_v.at[pl.ds(ci * 128, 128)],
                             out_hbm.at[idx_v.at[ci]], sem)
            for ci in range(8)
        ]
        for cp in cps:
            cp.wait()

    return k(memory, addr, rows)


def kernel(query, value, location_id, memory, Wq, bq, Wk, bk, Wg, bg):
    del bk
    addr = (location_id.astype(jnp.int32)) % NUM_SLOTS
    bq2 = bq.reshape(1, D)
    Wg2 = Wg.reshape(1, D)
    bg2 = bg.reshape(1, 1)
    addr_c = addr.reshape(B, 1)
    addr_r = addr.reshape(1, B)

    orig = _sc_gather(memory, addr)
    rows, q2 = _combine(value, Wg2, bg2, addr_c, addr_r, orig,
                        query, Wq, bq2, Wk)
    memT_aug = jnp.pad(
        jnp.concatenate(
            [memory.T,
             jnp.ones((1, NUM_SLOTS), jnp.float32),
             jnp.zeros((DA - D - 1, NUM_SLOTS), jnp.float32)], axis=0),
        ((0, 0), (0, MP - NUM_SLOTS)))
    attn_main, tail, rv = _attention(q2, memT_aug)
    attn = _tailmerge(attn_main, tail)
    new_memory = _sc_write(memory, addr, rows)[:NUM_SLOTS]
    return (rv, attn, new_memory)
